# baseline (speedup 1.0000x reference)
import functools

import jax
import jax.numpy as jnp
from jax import lax
from jax.experimental import pallas as pl
from jax.experimental.pallas import tpu as pltpu
from jax.experimental.pallas import tpu_sc as plsc

_CHUNK = 128
_NBUF = 2
_LANES = 16
_SUB = 8


@functools.lru_cache(maxsize=None)
def _build_gather(batch: int, num_rows: int, dim: int):
    info = plsc.get_sparse_core_info()
    nw = info.num_cores * info.num_subcores
    b_per_w = batch // nw
    n_chunks = b_per_w // _CHUNK
    assert batch % (nw * _CHUNK) == 0
    n_vec = _CHUNK // _LANES

    mesh = plsc.VectorSubcoreMesh(core_axis_name="c", subcore_axis_name="s")

    @functools.partial(
        pl.kernel,
        mesh=mesh,
        compiler_params=pltpu.CompilerParams(
            needs_layout_passes=False, skip_device_barrier=True
        ),
        out_type=jax.ShapeDtypeStruct((dim, batch), jnp.float32),
        scratch_types=[
            pltpu.VMEM((b_per_w,), jnp.int32),
            pltpu.VMEM((_NBUF, _CHUNK // _SUB, _SUB, dim), jnp.float32),
            pltpu.VMEM((_NBUF, dim, _CHUNK), jnp.float32),
            pltpu.SemaphoreType.DMA,
            pltpu.SemaphoreType.DMA,
        ],
    )
    def gather(tp3_hbm, idx_hbm, out_hbm, idx_v, bufs, tbufs, sem_g, sem_s):
        wid = lax.axis_index("s") * info.num_cores + lax.axis_index("c")
        base = wid * b_per_w
        pltpu.sync_copy(idx_hbm.at[pl.ds(base, b_per_w)], idx_v)
        lane = lax.iota(jnp.int32, _LANES)
        zero = lane * 0

        def out_col(c):
            return pl.multiple_of(base + c * _CHUNK, _CHUNK)

        def fire_chunk(c):
            buf = bufs.at[c % _NBUF]

            def group(g, carry):
                vec = idx_v[pl.ds(c * _CHUNK + g * _LANES, _LANES)]
                for l in range(_LANES):
                    j = g * _LANES + l
                    row = jnp.sum(jnp.where(lane == l, vec, 0))
                    pltpu.async_copy(
                        tp3_hbm.at[row >> 3, row & (_SUB - 1)],
                        buf.at[j // _SUB, j % _SUB],
                        sem_g,
                    )
                return carry

            lax.fori_loop(0, _CHUNK // _LANES, group, 0)

        def drain_chunk(c):
            buf = bufs.at[c % _NBUF]
            pltpu.make_async_copy(
                tp3_hbm.at[pl.ds(0, _CHUNK // _SUB)], buf, sem_g
            ).wait()

        def transpose_chunk(c):
            buf = bufs.at[c % _NBUF]
            tbuf = tbufs.at[c % _NBUF]

            def col(d, carry):
                dv = zero + d
                for k in range(n_vec):
                    jv = lane + k * _LANES
                    v = plsc.load_gather(
                        buf, [jv >> 3, jv & (_SUB - 1), dv]
                    )
                    tbuf[d, pl.ds(k * _LANES, _LANES)] = v
                return carry

            lax.fori_loop(0, dim, col, 0)

        def store_chunk(c):
            tbuf = tbufs.at[c % _NBUF]
            pltpu.async_copy(
                tbuf, out_hbm.at[:, pl.ds(out_col(c), _CHUNK)], sem_s
            )

        def wait_store(c):
            tbuf = tbufs.at[c % _NBUF]
            pltpu.make_async_copy(
                tbuf, out_hbm.at[:, pl.ds(out_col(c), _CHUNK)], sem_s
            ).wait()

        fire_chunk(0)
        for c in range(n_chunks):
            drain_chunk(c)
            if c + 1 < n_chunks:
                fire_chunk(c + 1)
            if c >= _NBUF:
                wait_store(c - _NBUF)
            transpose_chunk(c)
            store_chunk(c)
        for c in range(max(0, n_chunks - _NBUF), n_chunks):
            wait_store(c)

    return gather


def kernel(table, indices):
    batch = indices.shape[0]
    num_rows, dim = table.shape
    gather = _build_gather(batch, num_rows, dim)
    packed = jnp.reshape(table, (num_rows // _SUB, _SUB, dim))
    embs_t = gather(packed, indices)
    return (jnp.swapaxes(embs_t, 0, 1), indices)


# speedup vs baseline: 1.0569x; 1.0569x over previous
import functools

import jax
import jax.numpy as jnp
from jax import lax
from jax.experimental import pallas as pl
from jax.experimental.pallas import tpu as pltpu
from jax.experimental.pallas import tpu_sc as plsc

_CHUNK = 128
_NBUF = 2
_LANES = 16
_SUB = 8


@functools.lru_cache(maxsize=None)
def _build_gather(batch: int, num_rows: int, dim: int):
    info = plsc.get_sparse_core_info()
    nw = info.num_cores * info.num_subcores
    b_per_w = batch // nw
    n_chunks = b_per_w // _CHUNK
    assert batch % (nw * _CHUNK) == 0

    mesh = plsc.VectorSubcoreMesh(core_axis_name="c", subcore_axis_name="s")

    @functools.partial(
        pl.kernel,
        mesh=mesh,
        compiler_params=pltpu.CompilerParams(
            needs_layout_passes=False, skip_device_barrier=True
        ),
        out_type=jax.ShapeDtypeStruct((batch, dim), jnp.float32),
        scratch_types=[
            pltpu.VMEM((b_per_w,), jnp.int32),
            pltpu.VMEM((_NBUF, _CHUNK, dim), jnp.float32),
            pltpu.SemaphoreType.DMA,
            pltpu.SemaphoreType.DMA,
        ],
    )
    def gather(tp3_hbm, idx_hbm, out_hbm, idx_v, bufs, sem_g, sem_s):
        wid = lax.axis_index("s") * info.num_cores + lax.axis_index("c")
        base = wid * b_per_w
        pltpu.sync_copy(idx_hbm.at[pl.ds(base, b_per_w)], idx_v)
        lane = lax.iota(jnp.int32, _LANES)

        def fire_chunk(c):
            buf = bufs.at[c % _NBUF]

            def group(g, carry):
                vec = idx_v[pl.ds(c * _CHUNK + g * _LANES, _LANES)]
                for l in range(_LANES):
                    row = jnp.sum(jnp.where(lane == l, vec, 0))
                    pltpu.async_copy(
                        tp3_hbm.at[row >> 3, row & (_SUB - 1)],
                        buf.at[g * _LANES + l],
                        sem_g,
                    )
                return carry

            lax.fori_loop(0, _CHUNK // _LANES, group, 0)

        def drain_chunk(c):
            buf = bufs.at[c % _NBUF]
            pltpu.make_async_copy(
                out_hbm.at[pl.ds(0, _CHUNK)], buf, sem_g
            ).wait()

        def store_chunk(c):
            buf = bufs.at[c % _NBUF]
            pltpu.async_copy(
                buf, out_hbm.at[pl.ds(base + c * _CHUNK, _CHUNK)], sem_s
            )

        def wait_store(c):
            buf = bufs.at[c % _NBUF]
            pltpu.make_async_copy(
                buf, out_hbm.at[pl.ds(base + c * _CHUNK, _CHUNK)], sem_s
            ).wait()

        fire_chunk(0)
        for c in range(n_chunks):
            drain_chunk(c)
            store_chunk(c)
            if c + 1 < n_chunks:
                if c + 1 >= _NBUF:
                    wait_store(c + 1 - _NBUF)
                fire_chunk(c + 1)
        for c in range(max(0, n_chunks - _NBUF), n_chunks):
            wait_store(c)

    return gather


def kernel(table, indices):
    batch = indices.shape[0]
    num_rows, dim = table.shape
    gather = _build_gather(batch, num_rows, dim)
    packed = jnp.reshape(table, (num_rows // _SUB, _SUB, dim))
    embs = gather(packed, indices)
    return (embs, indices)
